# Initial kernel scaffold; baseline (speedup 1.0000x reference)
import jax
import jax.numpy as jnp
from jax.experimental import pallas as pl


def kernel(x, ff_W, ff_b, ff1_W, ff1_b, gate_W, gate_b):
    raise NotImplementedError("write your pallas kernel here")


# speedup vs baseline: 27.1609x; 27.1609x over previous
import functools

import jax
import jax.numpy as jnp
from jax.experimental import pallas as pl
from jax.experimental.pallas import tpu as pltpu

SEQ = 512
DM = 1024
NE = 6
TOPK = 4
TILE = 256


def _expert_dims(pl_e):
    step = pl_e // 2
    pn = int((SEQ - pl_e) / step + 1)
    return step, pn


def _compose_body(dims, *refs):
    wf_refs = refs[:NE]
    bf_ref, b1_ref, w1_ref, a_ref, m_ref = refs[NE:]
    rows = jax.lax.broadcasted_iota(jnp.int32, (SEQ + 1, DM), 0)
    cols = jax.lax.broadcasted_iota(jnp.int32, (SEQ + 1, DM), 1)
    for e, (pl_e, step, pn, dm) in enumerate(dims):
        b1row = jnp.where(rows == SEQ, b1_ref[e:e + 1, :], 0.0)
        if pl_e >= 24:
            m_ref[...] = jnp.zeros((SEQ + 1, DM), jnp.float32)
            wf = wf_refs[e][...]
            for n in range(pn):
                m_ref[n * step:n * step + pl_e, n * dm:(n + 1) * dm] = wf
            m_ref[SEQ:SEQ + 1, :] = bf_ref[e:e + 1, :]
            mblk = m_ref[...]
        else:
            n = cols // dm
            p = rows - n * step
            valid = (p >= 0) & (p < pl_e) & (cols < pn * dm)
            blk = jnp.where(rows == SEQ, bf_ref[e:e + 1, :], 0.0)
            for pp in range(pl_e):
                blk = blk + jnp.where(valid & (p == pp),
                                      wf_refs[e][pp:pp + 1, :], 0.0)
            mblk = blk
        a_ref[:, e * DM:(e + 1) * DM] = (
            jnp.dot(mblk, w1_ref[e], preferred_element_type=jnp.float32)
            + b1row)


def _main_body(x_ref, gw_ref, a_ref, o_ref):
    xb = x_ref[...]
    logits = jnp.dot(xb, gw_ref[...],
                     preferred_element_type=jnp.float32)
    col = [logits[:, j:j + 1] for j in range(NE)]
    keep = []
    for e in range(NE):
        rank = jnp.zeros_like(col[e])
        for j in range(NE):
            if j == e:
                continue
            beats = (col[j] > col[e]) if j > e else (col[j] >= col[e])
            rank = rank + beats.astype(jnp.float32)
        keep.append(rank < float(TOPK))
    mx = jnp.max(logits, axis=1, keepdims=True)
    pv = [jnp.where(keep[e], jnp.exp(col[e] - mx), 0.0) for e in range(NE)]
    denom = pv[0] + pv[1] + pv[2] + pv[3] + pv[4] + pv[5]
    inv = 1.0 / denom
    acc = jnp.zeros((TILE, DM), jnp.float32)
    for e in range(NE):
        ye = jnp.dot(xb, a_ref[:, e * DM:(e + 1) * DM],
                     preferred_element_type=jnp.float32)
        acc = acc + (pv[e] * inv) * ye
    o_ref[...] = acc


def kernel(x, ff_W, ff_b, ff1_W, ff1_b, gate_W, gate_b):
    B, V, S = x.shape
    T = B * V
    dims = []
    for e in range(NE):
        pl_e, dm = ff_W[e].shape
        step, pn = _expert_dims(pl_e)
        dims.append((pl_e, step, pn, dm))
    dims = tuple(dims)

    xf = x.reshape(T, S)
    x_aug = jnp.concatenate([xf, jnp.ones((T, 1), jnp.float32)], axis=1)
    gw_aug = jnp.concatenate([gate_W, gate_b[None, :]], axis=0)
    w1_pad = jnp.stack([
        jnp.pad(ff1_W[e], ((0, DM - ff1_W[e].shape[0]), (0, 0)))
        for e in range(NE)])
    bf_tiled = jnp.stack([
        jnp.pad(jnp.tile(ff_b[e], dims[e][2]),
                (0, DM - dims[e][2] * dims[e][3]))
        for e in range(NE)])
    b1_stack = jnp.stack(ff1_b)
    wf_args = []
    for e, (pl_e, step, pn, dm) in enumerate(dims):
        if pl_e >= 24:
            wf_args.append(ff_W[e])
        else:
            wf_args.append(jnp.pad(jnp.tile(ff_W[e], (1, pn)),
                                   ((0, 0), (0, DM - pn * dm))))

    full = lambda shape: pl.BlockSpec(shape, lambda: (0,) * len(shape))
    a_mat = pl.pallas_call(
        functools.partial(_compose_body, dims),
        out_shape=jax.ShapeDtypeStruct((S + 1, NE * DM), jnp.float32),
        in_specs=[full(w.shape) for w in wf_args]
        + [full((NE, DM)), full((NE, DM)), full((NE, DM, DM))],
        out_specs=full((S + 1, NE * DM)),
        scratch_shapes=[pltpu.VMEM((S + 1, DM), jnp.float32)],
    )(*wf_args, bf_tiled, b1_stack, w1_pad)

    grid = (T // TILE,)
    out = pl.pallas_call(
        _main_body,
        grid=grid,
        out_shape=jax.ShapeDtypeStruct((T, DM), jnp.float32),
        in_specs=[
            pl.BlockSpec((TILE, S + 1), lambda i: (i, 0)),
            pl.BlockSpec((S + 1, NE), lambda i: (0, 0)),
            pl.BlockSpec((S + 1, NE * DM), lambda i: (0, 0)),
        ],
        out_specs=pl.BlockSpec((TILE, DM), lambda i: (i, 0)),
    )(x_aug, gw_aug, a_mat)

    return out.reshape(B, V, DM)


# speedup vs baseline: 28.4493x; 1.0474x over previous
import functools

import jax
import jax.numpy as jnp
from jax.experimental import pallas as pl
from jax.experimental.pallas import tpu as pltpu

SEQ = 512
DM = 1024
NE = 6
TOPK = 4
TILE = 256


def _expert_dims(pl_e):
    step = pl_e // 2
    pn = int((SEQ - pl_e) / step + 1)
    return step, pn


def _compose_body(dims, *refs):
    wf_refs = refs[:NE]
    bf_ref, b1_ref, w1_ref, a_ref, m_ref = refs[NE:]
    rows = jax.lax.broadcasted_iota(jnp.int32, (SEQ + 1, DM), 0)
    cols = jax.lax.broadcasted_iota(jnp.int32, (SEQ + 1, DM), 1)
    for e, (pl_e, step, pn, dm) in enumerate(dims):
        b1row = jnp.where(rows == SEQ, b1_ref[e:e + 1, :].astype(jnp.float32),
                          0.0)
        if pl_e >= 24:
            m_ref[...] = jnp.zeros((SEQ + 1, DM), jnp.bfloat16)
            wf = wf_refs[e][...]
            for n in range(pn):
                m_ref[n * step:n * step + pl_e, n * dm:(n + 1) * dm] = wf
            m_ref[SEQ:SEQ + 1, :] = bf_ref[e:e + 1, :].astype(jnp.bfloat16)
            mblk = m_ref[...]
        else:
            n = cols // dm
            p = rows - n * step
            valid = (p >= 0) & (p < pl_e) & (cols < pn * dm)
            blk = jnp.where(rows == SEQ, bf_ref[e:e + 1, :].astype(jnp.float32),
                            0.0)
            for pp in range(pl_e):
                blk = blk + jnp.where(valid & (p == pp),
                                      wf_refs[e][pp:pp + 1,
                                                 :].astype(jnp.float32),
                                      0.0)
            mblk = blk.astype(jnp.bfloat16)
        a_ref[:, e * DM:(e + 1) * DM] = (
            jnp.dot(mblk, w1_ref[e], preferred_element_type=jnp.float32)
            + b1row).astype(jnp.bfloat16)


def _main_body(x_ref, x16_ref, gw_ref, a_ref, o_ref):
    xb = x_ref[...]
    xb16 = x16_ref[...]
    logits = jnp.dot(xb, gw_ref[...],
                     preferred_element_type=jnp.float32)
    col = [logits[:, j:j + 1] for j in range(NE)]
    keep = []
    for e in range(NE):
        rank = jnp.zeros_like(col[e])
        for j in range(NE):
            if j == e:
                continue
            beats = (col[j] > col[e]) if j > e else (col[j] >= col[e])
            rank = rank + beats.astype(jnp.float32)
        keep.append(rank < float(TOPK))
    mx = jnp.max(logits, axis=1, keepdims=True)
    pv = [jnp.where(keep[e], jnp.exp(col[e] - mx), 0.0) for e in range(NE)]
    denom = pv[0] + pv[1] + pv[2] + pv[3] + pv[4] + pv[5]
    inv = 1.0 / denom
    acc = jnp.zeros((TILE, DM), jnp.float32)
    for e in range(NE):
        ye = jnp.dot(xb16, a_ref[:, e * DM:(e + 1) * DM],
                     preferred_element_type=jnp.float32)
        acc = acc + (pv[e] * inv) * ye
    o_ref[...] = acc


def kernel(x, ff_W, ff_b, ff1_W, ff1_b, gate_W, gate_b):
    B, V, S = x.shape
    T = B * V
    dims = []
    for e in range(NE):
        pl_e, dm = ff_W[e].shape
        step, pn = _expert_dims(pl_e)
        dims.append((pl_e, step, pn, dm))
    dims = tuple(dims)

    xf = x.reshape(T, S)
    x_aug = jnp.concatenate([xf, jnp.ones((T, 1), jnp.float32)], axis=1)
    x_aug16 = x_aug.astype(jnp.bfloat16)
    gw_aug = jnp.concatenate([gate_W, gate_b[None, :]], axis=0)
    w1_pad = jnp.stack([
        jnp.pad(ff1_W[e], ((0, DM - ff1_W[e].shape[0]), (0, 0)))
        for e in range(NE)]).astype(jnp.bfloat16)
    bf_tiled = jnp.stack([
        jnp.pad(jnp.tile(ff_b[e], dims[e][2]),
                (0, DM - dims[e][2] * dims[e][3]))
        for e in range(NE)]).astype(jnp.bfloat16)
    b1_stack = jnp.stack(ff1_b)
    wf_args = []
    for e, (pl_e, step, pn, dm) in enumerate(dims):
        if pl_e >= 24:
            wf_args.append(ff_W[e].astype(jnp.bfloat16))
        else:
            wf_args.append(jnp.pad(jnp.tile(ff_W[e], (1, pn)),
                                   ((0, 0), (0, DM - pn * dm))
                                   ).astype(jnp.bfloat16))

    full = lambda shape: pl.BlockSpec(shape, lambda: (0,) * len(shape))
    a_mat = pl.pallas_call(
        functools.partial(_compose_body, dims),
        out_shape=jax.ShapeDtypeStruct((S + 1, NE * DM), jnp.bfloat16),
        in_specs=[full(w.shape) for w in wf_args]
        + [full((NE, DM)), full((NE, DM)), full((NE, DM, DM))],
        out_specs=full((S + 1, NE * DM)),
        scratch_shapes=[pltpu.VMEM((S + 1, DM), jnp.bfloat16)],
    )(*wf_args, bf_tiled, b1_stack, w1_pad)

    grid = (T // TILE,)
    out = pl.pallas_call(
        _main_body,
        grid=grid,
        out_shape=jax.ShapeDtypeStruct((T, DM), jnp.float32),
        in_specs=[
            pl.BlockSpec((TILE, S + 1), lambda i: (i, 0)),
            pl.BlockSpec((TILE, S + 1), lambda i: (i, 0)),
            pl.BlockSpec((S + 1, NE), lambda i: (0, 0)),
            pl.BlockSpec((S + 1, NE * DM), lambda i: (0, 0)),
        ],
        out_specs=pl.BlockSpec((TILE, DM), lambda i: (i, 0)),
    )(x_aug, x_aug16, gw_aug, a_mat)

    return out.reshape(B, V, DM)


# speedup vs baseline: 36.9056x; 1.2972x over previous
import functools

import jax
import jax.numpy as jnp
from jax.experimental import pallas as pl
from jax.experimental.pallas import tpu as pltpu

SEQ = 512
DM = 1024
NE = 6
TOPK = 4
TILE = 256


def _expert_dims(pl_e):
    step = pl_e // 2
    pn = int((SEQ - pl_e) / step + 1)
    return step, pn


def _compose_body(dims, *refs):
    wf_refs = refs[:NE]
    w1_refs = refs[NE:2 * NE]
    bf_ref, b1_ref, a_ref, m_ref, w1s_ref = refs[2 * NE:]
    rows = jax.lax.broadcasted_iota(jnp.int32, (SEQ + 1, DM), 0)
    cols = jax.lax.broadcasted_iota(jnp.int32, (SEQ + 1, DM), 1)
    for e, (pl_e, step, pn, dm) in enumerate(dims):
        b1row = jnp.where(rows == SEQ, b1_ref[e:e + 1, :], 0.0)
        if pl_e >= 24:
            m_ref[...] = jnp.zeros((SEQ + 1, DM), jnp.bfloat16)
            wf = wf_refs[e][...]
            for n in range(pn):
                m_ref[n * step:n * step + pl_e, n * dm:(n + 1) * dm] = wf
            m_ref[SEQ:SEQ + 1, :] = bf_ref[e:e + 1, :]
            mblk = m_ref[...]
        else:
            n = cols // dm
            p = jnp.where((cols < pn * dm) & (rows < SEQ),
                          rows - n * step, -1)
            blk = jnp.where(rows == SEQ,
                            bf_ref[e:e + 1, :].astype(jnp.float32), 0.0)
            for pp in range(pl_e):
                blk = blk + jnp.where(p == pp,
                                      wf_refs[e][pp:pp + 1,
                                                 :].astype(jnp.float32),
                                      0.0)
            mblk = blk.astype(jnp.bfloat16)
        nk = pn * dm
        w1s_ref[0:nk, :] = w1_refs[e][...].astype(jnp.bfloat16)
        if nk < DM:
            w1s_ref[nk:DM, :] = jnp.zeros((DM - nk, DM), jnp.bfloat16)
        a_ref[:, e * DM:(e + 1) * DM] = (
            jnp.dot(mblk, w1s_ref[...], preferred_element_type=jnp.float32)
            + b1row).astype(jnp.bfloat16)


def _main_body(x_ref, gw_ref, gb_ref, a_ref, o_ref):
    xb = x_ref[...]
    xb16 = xb.astype(jnp.bfloat16)
    logits = jnp.dot(xb, gw_ref[...],
                     preferred_element_type=jnp.float32) + gb_ref[...]
    col = [logits[:, j:j + 1] for j in range(NE)]
    keep = []
    for e in range(NE):
        rank = jnp.zeros_like(col[e])
        for j in range(NE):
            if j == e:
                continue
            beats = (col[j] > col[e]) if j > e else (col[j] >= col[e])
            rank = rank + beats.astype(jnp.float32)
        keep.append(rank < float(TOPK))
    mx = jnp.max(logits, axis=1, keepdims=True)
    pv = [jnp.where(keep[e], jnp.exp(col[e] - mx), 0.0) for e in range(NE)]
    denom = pv[0] + pv[1] + pv[2] + pv[3] + pv[4] + pv[5]
    inv = 1.0 / denom
    acc = jnp.zeros((TILE, DM), jnp.float32)
    for e in range(NE):
        ye = jnp.dot(xb16, a_ref[0:SEQ, e * DM:(e + 1) * DM],
                     preferred_element_type=jnp.float32)
        ye = ye + a_ref[SEQ:SEQ + 1, e * DM:(e + 1) * DM].astype(jnp.float32)
        acc = acc + (pv[e] * inv) * ye
    o_ref[...] = acc


def kernel(x, ff_W, ff_b, ff1_W, ff1_b, gate_W, gate_b):
    B, V, S = x.shape
    T = B * V
    dims = []
    for e in range(NE):
        pl_e, dm = ff_W[e].shape
        step, pn = _expert_dims(pl_e)
        dims.append((pl_e, step, pn, dm))
    dims = tuple(dims)

    xf = x.reshape(T, S)
    bf_tiled = jnp.stack([
        jnp.pad(jnp.tile(ff_b[e], dims[e][2]),
                (0, DM - dims[e][2] * dims[e][3]))
        for e in range(NE)]).astype(jnp.bfloat16)
    b1_stack = jnp.stack(ff1_b)
    wf_args = []
    for e, (pl_e, step, pn, dm) in enumerate(dims):
        if pl_e >= 24:
            wf_args.append(ff_W[e].astype(jnp.bfloat16))
        else:
            wf_args.append(jnp.pad(jnp.tile(ff_W[e], (1, pn)),
                                   ((0, 0), (0, DM - pn * dm))))

    full = lambda shape: pl.BlockSpec(shape, lambda: (0,) * len(shape))
    a_mat = pl.pallas_call(
        functools.partial(_compose_body, dims),
        out_shape=jax.ShapeDtypeStruct((SEQ + 1, NE * DM), jnp.bfloat16),
        in_specs=[full(w.shape) for w in wf_args]
        + [full(w.shape) for w in ff1_W]
        + [full((NE, DM)), full((NE, DM))],
        out_specs=full((SEQ + 1, NE * DM)),
        scratch_shapes=[pltpu.VMEM((SEQ + 1, DM), jnp.bfloat16),
                        pltpu.VMEM((DM, DM), jnp.bfloat16)],
    )(*wf_args, *ff1_W, bf_tiled, b1_stack)

    grid = (T // TILE,)
    out = pl.pallas_call(
        _main_body,
        grid=grid,
        out_shape=jax.ShapeDtypeStruct((T, DM), jnp.float32),
        in_specs=[
            pl.BlockSpec((TILE, S), lambda i: (i, 0)),
            pl.BlockSpec((S, NE), lambda i: (0, 0)),
            pl.BlockSpec((1, NE), lambda i: (0, 0)),
            pl.BlockSpec((SEQ + 1, NE * DM), lambda i: (0, 0)),
        ],
        out_specs=pl.BlockSpec((TILE, DM), lambda i: (i, 0)),
    )(xf, gate_W, gate_b[None, :], a_mat)

    return out.reshape(B, V, DM)
